# baseline (speedup 1.0000x reference)
import jax
import jax.numpy as jnp
from jax import lax
from jax.experimental import pallas as pl
from jax.experimental.pallas import tpu as pltpu
from jax.experimental.pallas import tpu_sc as plsc

_ROWS = 128
_N = 32768
_K = 64
_L = 16
_NVEC = _N // _L
_NC = 2
_NS = 16
_NW = _NC * _NS
_RPW = _ROWS // _NW


def _popcount(mask):
    r = plsc.all_reduce_population_count(mask)
    return r[0] if r.ndim else r


def _lsr(x, n):
    return lax.shift_right_logical(x, jnp.int32(n))


def _scan_hist(hist, kneed, chunk_of_s):
    lanes = lax.iota(jnp.int32, _L)
    zeros = jnp.zeros((_L,), jnp.int32)

    def step(s, carry):
        acc, found, bstar, kp = carry
        base, cdesc = chunk_of_s(s)
        v = hist[pl.ds(base, _L)]
        hist[pl.ds(base, _L)] = zeros
        cdv = jnp.broadcast_to(cdesc, (_L,))
        vv = jnp.where(cdv, jnp.flip(v, 0), v)
        cs = jnp.cumsum(vv)
        i0 = _popcount(acc + cs < kneed)
        hit = i0 < _L
        cs_prev = jnp.sum(jnp.where(lanes == i0 - 1, cs, 0))
        upd = jnp.logical_and(hit, found == 0)
        bsel = jnp.where(cdesc, base + _L - 1 - i0, base + i0)
        bstar = jnp.where(upd, bsel, bstar)
        kp = jnp.where(upd, kneed - acc - cs_prev, kp)
        found = jnp.where(hit, jnp.int32(1), found)
        acc = acc + cs[_L - 1]
        return acc, found, bstar, kp

    init = (jnp.int32(0), jnp.int32(0), jnp.int32(0), jnp.int32(0))
    _, _, bstar, kp = lax.fori_loop(0, _L, step, init)
    return bstar, kp


def _sc_body(x_hbm, out_hbm, buf0, buf1, cand, hist, si0, si1, so0, so1):
    lanes = lax.iota(jnp.int32, _L)
    ones = jnp.ones((_L,), jnp.int32)
    wid = lax.axis_index("s") * _NC + lax.axis_index("c")
    row0 = wid * _RPW

    def clr(i, c):
        hist[pl.ds(i * _L, _L)] = jnp.zeros((_L,), jnp.int32)
        return c
    lax.fori_loop(0, 16, clr, 0)

    def top_byte_order(s):
        pos = s < 8
        base = jnp.where(pos, (7 - s) * _L, s * _L)
        return base, pos

    def process(buf, row):
        @plsc.parallel_loop(0, _NVEC, unroll=8)
        def p_hist3(i):
            u = lax.bitcast_convert_type(buf[pl.ds(i * _L, _L)], jnp.int32)
            plsc.addupdate_scatter(hist, [_lsr(u, 24)], ones)

        b3, kneed = _scan_hist(hist, jnp.int32(_K), top_byte_order)
        desc = b3 < 128

        def low_byte_order(s):
            base = jnp.where(desc, (15 - s) * _L, s * _L)
            return base, desc

        @plsc.parallel_loop(0, _NVEC, unroll=4, carry=jnp.zeros((_L,), jnp.int32))
        def p_compact3(i, offv):
            u = lax.bitcast_convert_type(buf[pl.ds(i * _L, _L)], jnp.int32)
            msk = _lsr(u, 24) == b3
            mi = msk.astype(jnp.int32)
            excl = plsc.cumsum(mi) - mi
            plsc.store_scatter(cand, [offv + excl], i * _L + lanes, mask=msk)
            return offv + plsc.all_reduce_population_count(msk)

        m = p_compact3[0]

        def level(shift, m, kneed):
            nv = (m + _L - 1) // _L

            def p_hist(i, c):
                pos = cand[pl.ds(i * _L, _L)]
                valid = (i * _L + lanes) < m
                u = lax.bitcast_convert_type(
                    plsc.load_gather(buf, [pos], mask=valid), jnp.int32)
                b = _lsr(u, shift) & 255
                plsc.addupdate_scatter(hist, [b], ones, mask=valid)
                return c

            lax.fori_loop(0, nv, p_hist, 0)
            bs, kneed = _scan_hist(hist, kneed, low_byte_order)

            def p_compact(i, offv):
                pos = cand[pl.ds(i * _L, _L)]
                valid = (i * _L + lanes) < m
                u = lax.bitcast_convert_type(
                    plsc.load_gather(buf, [pos], mask=valid), jnp.int32)
                msk = jnp.logical_and(valid, (_lsr(u, shift) & 255) == bs)
                mi = msk.astype(jnp.int32)
                excl = plsc.cumsum(mi) - mi
                plsc.store_scatter(cand, [offv + excl], pos, mask=msk)
                return offv + plsc.all_reduce_population_count(msk)

            m2 = lax.fori_loop(0, nv, p_compact, jnp.zeros((_L,), jnp.int32))[0]
            return bs, m2, kneed

        b2, m, kneed = level(16, m, kneed)
        b1, m, kneed = level(8, m, kneed)
        b0, m, kneed = level(0, m, kneed)
        b3s = jnp.where(b3 >= 128, b3 - 256, b3)
        t = ((b3s * 256 + b2) * 256 + b1) * 256 + b0
        mfin = kneed
        tmax = jnp.maximum(t, jnp.int32(0))

        @plsc.parallel_loop(0, _NVEC, unroll=8)
        def p_final(i):
            u = lax.bitcast_convert_type(buf[pl.ds(i * _L, _L)], jnp.int32)
            buf[pl.ds(i * _L, _L)] = jnp.where(
                u > tmax, lax.bitcast_convert_type(u, jnp.float32), jnp.float32(0))

        tf = jnp.broadcast_to(lax.bitcast_convert_type(t, jnp.float32), (_L,))
        nvt = (mfin + _L - 1) // _L

        def p_tie(i, c):
            pos = cand[pl.ds(i * _L, _L)]
            msk = jnp.logical_and((i * _L + lanes) < mfin, t > 0)
            plsc.store_scatter(buf, [pos], tf, mask=msk)
            return c

        lax.fori_loop(0, nvt, p_tie, 0)

    bufs = (buf0, buf1)
    sin = (si0, si1)
    sout = (so0, so1)
    in_h = [None] * _RPW
    out_h = [None] * _RPW
    in_h[0] = pltpu.async_copy(x_hbm.at[row0], buf0, si0)
    for r in range(_RPW):
        b = bufs[r % 2]
        if r + 1 < _RPW:
            if r >= 1:
                out_h[r - 1].wait()
            in_h[r + 1] = pltpu.async_copy(
                x_hbm.at[row0 + r + 1], bufs[(r + 1) % 2], sin[(r + 1) % 2])
        in_h[r].wait()
        process(b, row0 + r)
        out_h[r] = pltpu.async_copy(b, out_hbm.at[row0 + r], sout[r % 2])
    out_h[_RPW - 2].wait()
    out_h[_RPW - 1].wait()


@jax.jit
def kernel(x):
    mesh = plsc.VectorSubcoreMesh(core_axis_name="c", subcore_axis_name="s")
    run = pl.kernel(
        _sc_body,
        out_type=jax.ShapeDtypeStruct((_ROWS, _N), jnp.float32),
        mesh=mesh,
        scratch_types=[
            pltpu.VMEM((_N,), jnp.float32),
            pltpu.VMEM((_N,), jnp.float32),
            pltpu.VMEM((_N + _L,), jnp.int32),
            pltpu.VMEM((256,), jnp.int32),
            pltpu.SemaphoreType.DMA,
            pltpu.SemaphoreType.DMA,
            pltpu.SemaphoreType.DMA,
            pltpu.SemaphoreType.DMA,
        ],
        compiler_params=pltpu.CompilerParams(needs_layout_passes=False),
    )
    return run(x)


# speedup vs baseline: 1.6407x; 1.6407x over previous
import jax
import jax.numpy as jnp
from jax import lax
from jax.experimental import pallas as pl
from jax.experimental.pallas import tpu as pltpu
from jax.experimental.pallas import tpu_sc as plsc

_ROWS = 128
_N = 32768
_K = 64
_L = 16
_NVEC = _N // _L
_NC = 2
_NS = 16
_NW = _NC * _NS
_RPW = _ROWS // _NW
_NFINE = 4096


def _keyify(v):
    u = lax.bitcast_convert_type(v, jnp.int32)
    return jnp.where(u >= 0, u, u ^ jnp.int32(0x7FFFFFFF))


def _popcount(mask):
    r = plsc.all_reduce_population_count(mask)
    return r[0] if r.ndim else r


def _chunk_step(hist, base, acc, kneed, lanes, zeros):
    v = hist[pl.ds(base, _L)]
    hist[pl.ds(base, _L)] = zeros
    cs = jnp.cumsum(jnp.flip(v, 0))
    i0 = _popcount(acc + cs < kneed)
    cs_prev = jnp.sum(jnp.where(lanes == i0 - 1, cs, 0))
    return cs, i0, cs_prev


def _scan_desc(hist, kneed, nchunks):
    lanes = lax.iota(jnp.int32, _L)
    zeros = jnp.zeros((_L,), jnp.int32)

    def step(c, carry):
        acc, found, bstar, kp = carry
        base = (nchunks - 1 - c) * _L
        cs, i0, cs_prev = _chunk_step(hist, base, acc, kneed, lanes, zeros)
        hit = i0 < _L
        upd = jnp.logical_and(hit, found == 0)
        bstar = jnp.where(upd, base + _L - 1 - i0, bstar)
        kp = jnp.where(upd, kneed - acc - cs_prev, kp)
        found = jnp.where(hit, jnp.int32(1), found)
        acc = acc + cs[_L - 1]
        return acc, found, bstar, kp

    init = (jnp.int32(0), jnp.int32(0), jnp.int32(0), jnp.int32(0))
    _, _, bstar, kp = lax.fori_loop(0, nchunks, step, init)
    return bstar, kp


def _scan_fine(hist, kneed, cstart):
    lanes = lax.iota(jnp.int32, _L)
    zeros = jnp.zeros((_L,), jnp.int32)

    def cond(carry):
        return carry[1] == 0

    def body(carry):
        c, found, acc, bstar, kp = carry
        base = c * _L
        cs, i0, cs_prev = _chunk_step(hist, base, acc, kneed, lanes, zeros)
        hit = i0 < _L
        bstar = jnp.where(hit, base + _L - 1 - i0, bstar)
        kp = jnp.where(hit, kneed - acc - cs_prev, kp)
        found = jnp.where(hit, jnp.int32(1), found)
        return c - 1, found, acc + cs[_L - 1], bstar, kp

    init = (cstart, jnp.int32(0), jnp.int32(0), jnp.int32(0), jnp.int32(0))
    _, _, _, bstar, kp = lax.while_loop(cond, body, init)
    return bstar, kp


def _sc_body(x_hbm, out_hbm, buf0, buf1, cand, histf, histc, si0, si1, so0, so1):
    lanes = lax.iota(jnp.int32, _L)
    ones = jnp.ones((_L,), jnp.int32)
    zeros = jnp.zeros((_L,), jnp.int32)
    wid = lax.axis_index("s") * _NC + lax.axis_index("c")
    row0 = wid * _RPW

    def clrf(i, c):
        histf[pl.ds(i * _L, _L)] = zeros
        return c
    lax.fori_loop(0, _NFINE // _L, clrf, 0)

    def clrc(i, c):
        histc[pl.ds(i * _L, _L)] = zeros
        return c
    lax.fori_loop(0, 16, clrc, 0)

    def process(buf, row):
        neg_inf = jnp.full((_L,), jnp.int32(-2147483647 - 1))

        @plsc.parallel_loop(0, _NVEC, unroll=8, carry=neg_inf)
        def p_hist(i, mx):
            k = _keyify(buf[pl.ds(i * _L, _L)])
            plsc.addupdate_scatter(histf, [(k >> 20) + 2048], ones)
            return jnp.maximum(mx, k)

        kmax = jnp.max(p_hist)
        cstart = ((kmax >> 20) + 2048) >> 4
        fb, kneed = _scan_fine(histf, jnp.int32(_K), cstart)
        p20 = fb - 2048
        klo = jnp.maximum((p20 << 20) - 1, jnp.int32(0))

        @plsc.parallel_loop(0, _NFINE // _L, unroll=8)
        def p_clr(i):
            histf[pl.ds(i * _L, _L)] = zeros

        @plsc.parallel_loop(0, _NVEC, unroll=4, carry=jnp.int32(0))
        def p_main(i, off):
            v = buf[pl.ds(i * _L, _L)]
            k = _keyify(v)
            buf[pl.ds(i * _L, _L)] = jnp.where(k > klo, v, jnp.float32(0))
            msk = (k >> 20) == p20
            plsc.store_compressed(cand.at[pl.ds(off, _L)], i * _L + lanes, mask=msk)
            return off + _popcount(msk)

        m = p_main

        nv = (m + _L - 1) // _L

        def level(prefix, pshift, shift, bmask, nchunks, kneed):
            def ph(i, c):
                pos = cand[pl.ds(i * _L, _L)]
                valid = (i * _L + lanes) < m
                k = _keyify(plsc.load_gather(buf, [pos], mask=valid))
                ok = jnp.logical_and(valid, (k >> pshift) == prefix)
                plsc.addupdate_scatter(histc, [(k >> shift) & bmask], ones, mask=ok)
                return c

            lax.fori_loop(0, nv, ph, 0)
            bs, kneed2 = _scan_desc(histc, kneed, nchunks)
            return bs, kneed2

        b1, kneed = level(p20, 20, 12, 255, 16, kneed)
        p12 = p20 * 256 + b1
        b2, kneed = level(p12, 12, 4, 255, 16, kneed)
        p4 = p12 * 256 + b2
        b3v, kneed = level(p4, 4, 0, 15, 1, kneed)
        t = p4 * 16 + b3v
        mfin = kneed
        tmax = jnp.maximum(t, jnp.int32(0))

        def p_fix(i, eq_seen):
            pos = cand[pl.ds(i * _L, _L)]
            valid = (i * _L + lanes) < m
            v = plsc.load_gather(buf, [pos], mask=valid)
            k = _keyify(v)
            eq = jnp.logical_and(k == t, valid)
            eqc = jnp.cumsum(eq.astype(jnp.int32))
            sel = jnp.logical_or(k > tmax,
                                 jnp.logical_and(eq, eq_seen + eqc <= mfin))
            outv = jnp.where(jnp.logical_and(sel, k > 0), v, jnp.float32(0))
            plsc.store_scatter(buf, [pos], outv, mask=valid)
            return eq_seen + plsc.all_reduce_population_count(eq)

        lax.fori_loop(0, nv, p_fix, jnp.zeros((_L,), jnp.int32))

    bufs = (buf0, buf1)
    sin = (si0, si1)
    sout = (so0, so1)
    in_h = [None] * _RPW
    out_h = [None] * _RPW
    in_h[0] = pltpu.async_copy(x_hbm.at[row0], buf0, si0)
    for r in range(_RPW):
        b = bufs[r % 2]
        if r + 1 < _RPW:
            if r >= 1:
                out_h[r - 1].wait()
            in_h[r + 1] = pltpu.async_copy(
                x_hbm.at[row0 + r + 1], bufs[(r + 1) % 2], sin[(r + 1) % 2])
        in_h[r].wait()
        process(b, row0 + r)
        out_h[r] = pltpu.async_copy(b, out_hbm.at[row0 + r], sout[r % 2])
    out_h[_RPW - 2].wait()
    out_h[_RPW - 1].wait()


@jax.jit
def kernel(x):
    mesh = plsc.VectorSubcoreMesh(core_axis_name="c", subcore_axis_name="s")
    run = pl.kernel(
        _sc_body,
        out_type=jax.ShapeDtypeStruct((_ROWS, _N), jnp.float32),
        mesh=mesh,
        scratch_types=[
            pltpu.VMEM((_N,), jnp.float32),
            pltpu.VMEM((_N,), jnp.float32),
            pltpu.VMEM((_N + _L,), jnp.int32),
            pltpu.VMEM((_NFINE,), jnp.int32),
            pltpu.VMEM((256,), jnp.int32),
            pltpu.SemaphoreType.DMA,
            pltpu.SemaphoreType.DMA,
            pltpu.SemaphoreType.DMA,
            pltpu.SemaphoreType.DMA,
        ],
        compiler_params=pltpu.CompilerParams(needs_layout_passes=False),
    )
    return run(x)
